# baseline (speedup 1.0000x reference)
import functools

import jax
import jax.numpy as jnp
from jax import lax
from jax.experimental import pallas as pl
from jax.experimental.pallas import tpu as pltpu
from jax.experimental.pallas import tpu_sc as plsc

N = 10000
E = 320000
D = 128

NC = 2
NS = 16
NW = NC * NS

ACC_R = 10240
SL = ACC_R // NS
AGG_R = 10112
ASL = AGG_R // NS

C = 128
CH = 81
EPT = C * CH
E_PAD = NW * EPT
ED = E // NW


@functools.cache
def _mesh():
    return plsc.VectorSubcoreMesh(
        core_axis_name="c", subcore_axis_name="s", num_cores=NC, num_subcores=NS
    )


_sc_params = pltpu.CompilerParams(needs_layout_passes=False)


def _deg_body(edge, parts, hist0, hist1, idx0, idx1, hist_sh, rbuf, obuf):
    c = lax.axis_index("c")
    s = lax.axis_index("s")
    g = c * NS + s
    zeros16 = jnp.zeros((16,), jnp.float32)
    ones16 = jnp.ones((16,), jnp.float32)
    hists = (hist0, hist1)
    idxs = (idx0, idx1)

    def zero_body(i, _):
        hist0[pl.ds(i * 16, 16)] = zeros16
        hist1[pl.ds(i * 16, 16)] = zeros16
        return _

    lax.fori_loop(0, ACC_R // 16, zero_body, None)

    for k in range(2):
        pltpu.sync_copy(edge.at[pl.ds(k * E + g * ED, ED)], idxs[k])

        def hist_body(j, _, k=k):
            iv = idxs[k][pl.ds(j * 16, 16)]
            plsc.addupdate_scatter(hists[k], [iv], ones16)
            return _

        lax.fori_loop(0, ED // 16, hist_body, None)

    for k in range(2):
        pltpu.sync_copy(hists[k], hist_sh.at[pl.ds((s * 2 + k) * ACC_R, ACC_R)])
    plsc.subcore_barrier()

    for k in range(2):
        for t in range(NS):
            pltpu.sync_copy(
                hist_sh.at[pl.ds((t * 2 + k) * ACC_R + s * SL, SL)],
                rbuf.at[pl.ds(t * SL, SL)],
            )

        def red_body(j, _):
            acc = rbuf[pl.ds(j * 16, 16)]
            for t in range(1, NS):
                acc = acc + rbuf[pl.ds(t * SL + j * 16, 16)]
            obuf[pl.ds(j * 16, 16)] = acc
            return _

        lax.fori_loop(0, SL // 16, red_body, None)
        pltpu.sync_copy(obuf, parts.at[pl.ds((c * 2 + k) * ACC_R + s * SL, SL)])


def _deg_call(edge_flat):
    return pl.kernel(
        _deg_body,
        out_type=jax.ShapeDtypeStruct((NC * 2 * ACC_R,), jnp.float32),
        mesh=_mesh(),
        compiler_params=_sc_params,
        scratch_types=[
            pltpu.VMEM((ACC_R,), jnp.float32),
            pltpu.VMEM((ACC_R,), jnp.float32),
            pltpu.VMEM((ED,), jnp.int32),
            pltpu.VMEM((ED,), jnp.int32),
            pltpu.VMEM_SHARED((NS * 2 * ACC_R,), jnp.float32),
            pltpu.VMEM((NS * SL,), jnp.float32),
            pltpu.VMEM((SL,), jnp.float32),
        ],
    )(edge_flat)


_NB = 3


def _agg_body(table, srcp, dstp, zeros, y, acc, *bufs):
    si = bufs[0:_NB]
    di = bufs[_NB:2 * _NB]
    rows = bufs[2 * _NB:3 * _NB]
    gs = bufs[3 * _NB:4 * _NB]
    ss = bufs[4 * _NB:5 * _NB]
    c = lax.axis_index("c")
    s = lax.axis_index("s")

    pltpu.sync_copy(zeros.at[pl.ds(s * ASL, ASL)], acc.at[pl.ds(s * ASL, ASL)])
    plsc.subcore_barrier()

    base = (c * NS + s) * EPT

    def load_idx(g, j):
        e0 = pl.multiple_of(base + g * C, 8)
        pltpu.sync_copy(srcp.at[pl.ds(e0, C)], si[j])
        pltpu.sync_copy(dstp.at[pl.ds(e0, C)], di[j])

    for j in range(_NB):
        load_idx(j, j)
        pltpu.async_copy(table.at[si[j]], rows[j], gs[j])

    def triad_body(q, _):
        descs = []
        for j in range(_NB):
            pltpu.make_async_copy(table.at[si[j]], rows[j], gs[j]).wait()
            descs.append(pltpu.async_copy(rows[j], acc.at[di[j]], ss[j], add=True))
        g0 = jnp.minimum((q + 1) * _NB, CH - _NB)
        for j in range(_NB):
            descs[j].wait()
            load_idx(g0 + j, j)
            pltpu.async_copy(table.at[si[j]], rows[j], gs[j])
        return _

    lax.fori_loop(0, CH // _NB, triad_body, None)

    for j in range(_NB):
        pltpu.make_async_copy(table.at[si[j]], rows[j], gs[j]).wait()

    plsc.subcore_barrier()
    pltpu.sync_copy(acc.at[pl.ds(s * ASL, ASL)], y.at[c, pl.ds(s * ASL, ASL)])


def _agg_call(table, srcp, dstp, zeros):
    return pl.kernel(
        _agg_body,
        out_type=jax.ShapeDtypeStruct((NC, AGG_R, D), jnp.float32),
        mesh=_mesh(),
        compiler_params=_sc_params,
        scratch_types=[
            pltpu.VMEM_SHARED((AGG_R, D), jnp.float32),
            *[pltpu.VMEM((C,), jnp.int32) for _ in range(_NB)],
            *[pltpu.VMEM((C,), jnp.int32) for _ in range(_NB)],
            *[pltpu.VMEM((C, D), jnp.float32) for _ in range(_NB)],
            *[pltpu.SemaphoreType.DMA for _ in range(_NB)],
            *[pltpu.SemaphoreType.DMA for _ in range(_NB)],
        ],
    )(table, srcp, dstp, zeros)


_ROWS_BLK = 1000


def _mm_body(x_ref, w_ref, o_ref):
    o_ref[...] = x_ref[...] @ w_ref[...]


def _tc1_body(m_ref, no_ref, o_ref):
    o_ref[...] = m_ref[...] * no_ref[...]


def _tc2_body(y_ref, ni_ref, no_ref, b1_ref, w2_ref, o_ref):
    yb = y_ref[0] + y_ref[1]
    h = jnp.maximum(yb * ni_ref[...] + b1_ref[...], 0.0)
    o_ref[...] = (h * no_ref[...]) @ w2_ref[...]


def _tc3_body(y_ref, ni_ref, b2_ref, o_ref):
    yb = y_ref[0] + y_ref[1]
    o_ref[...] = yb * ni_ref[...] + b2_ref[...]


def _mm(x, W1):
    return pl.pallas_call(
        _mm_body,
        out_shape=jax.ShapeDtypeStruct((N, D), jnp.float32),
        grid=(N // _ROWS_BLK,),
        in_specs=[
            pl.BlockSpec((_ROWS_BLK, D), lambda i: (i, 0)),
            pl.BlockSpec((D, D), lambda i: (0, 0)),
        ],
        out_specs=pl.BlockSpec((_ROWS_BLK, D), lambda i: (i, 0)),
    )(x, W1)


def _tc1(m, no2):
    return pl.pallas_call(
        _tc1_body,
        out_shape=jax.ShapeDtypeStruct((N, D), jnp.float32),
        grid=(N // _ROWS_BLK,),
        in_specs=[
            pl.BlockSpec((_ROWS_BLK, D), lambda i: (i, 0)),
            pl.BlockSpec((_ROWS_BLK, 1), lambda i: (i, 0)),
        ],
        out_specs=pl.BlockSpec((_ROWS_BLK, D), lambda i: (i, 0)),
    )(m, no2)


def _tc2(y1, ni2, no2, b1, W2):
    return pl.pallas_call(
        _tc2_body,
        out_shape=jax.ShapeDtypeStruct((N, D), jnp.float32),
        grid=(N // _ROWS_BLK,),
        in_specs=[
            pl.BlockSpec((NC, _ROWS_BLK, D), lambda i: (0, i, 0)),
            pl.BlockSpec((_ROWS_BLK, 1), lambda i: (i, 0)),
            pl.BlockSpec((_ROWS_BLK, 1), lambda i: (i, 0)),
            pl.BlockSpec((1, D), lambda i: (0, 0)),
            pl.BlockSpec((D, D), lambda i: (0, 0)),
        ],
        out_specs=pl.BlockSpec((_ROWS_BLK, D), lambda i: (i, 0)),
    )(y1, ni2, no2, b1, W2)


def _tc3(y2, ni2, b2):
    return pl.pallas_call(
        _tc3_body,
        out_shape=jax.ShapeDtypeStruct((N, D), jnp.float32),
        grid=(N // _ROWS_BLK,),
        in_specs=[
            pl.BlockSpec((NC, _ROWS_BLK, D), lambda i: (0, i, 0)),
            pl.BlockSpec((_ROWS_BLK, 1), lambda i: (i, 0)),
            pl.BlockSpec((1, D), lambda i: (0, 0)),
        ],
        out_specs=pl.BlockSpec((_ROWS_BLK, D), lambda i: (i, 0)),
    )(y2, ni2, b2)


@jax.jit
def kernel(x, edge_index, W1, b1, W2, b2):
    src = edge_index[0]
    dst = edge_index[1]

    mm1 = _mm(x, W1)
    parts = _deg_call(edge_index.reshape(-1)).reshape(NC, 2, ACC_R)
    deg = parts[0] + parts[1]
    norm_out2 = lax.rsqrt(jnp.clip(deg[0, :N], 1.0, None))[:, None]
    norm_in2 = lax.rsqrt(jnp.clip(deg[1, :N], 1.0, None))[:, None]

    pad = jnp.arange(E_PAD - E, dtype=jnp.int32) % 16
    srcp = jnp.concatenate([src, pad])
    dstp = jnp.concatenate([dst, N + pad])
    zeros = jnp.zeros((AGG_R, D), jnp.float32)

    t1 = _tc1(mm1, norm_out2)
    y1 = _agg_call(t1, srcp, dstp, zeros)

    t2 = _tc2(y1, norm_in2, norm_out2, b1[None, :], W2)
    y2 = _agg_call(t2, srcp, dstp, zeros)

    return _tc3(y2, norm_in2, b2[None, :])


# speedup vs baseline: 1.0384x; 1.0384x over previous
import functools

import jax
import jax.numpy as jnp
from jax import lax
from jax.experimental import pallas as pl
from jax.experimental.pallas import tpu as pltpu
from jax.experimental.pallas import tpu_sc as plsc

N = 10000
E = 320000
D = 128

NC = 2
NS = 16
NW = NC * NS

ACC_R = 10240
SL = ACC_R // NS
AGG_R = 10112
ASL = AGG_R // NS

C = 128
CH = 81
EPT = C * CH
E_PAD = NW * EPT
ED = E // NW


@functools.cache
def _mesh():
    return plsc.VectorSubcoreMesh(
        core_axis_name="c", subcore_axis_name="s", num_cores=NC, num_subcores=NS
    )


_sc_params = pltpu.CompilerParams(needs_layout_passes=False)


def _deg_body(edge, parts, hist0, hist1, idx0, idx1, hist_sh, rbuf, obuf):
    c = lax.axis_index("c")
    s = lax.axis_index("s")
    g = c * NS + s
    zeros16 = jnp.zeros((16,), jnp.float32)
    ones16 = jnp.ones((16,), jnp.float32)
    hists = (hist0, hist1)
    idxs = (idx0, idx1)

    def zero_body(i, _):
        hist0[pl.ds(i * 16, 16)] = zeros16
        hist1[pl.ds(i * 16, 16)] = zeros16
        return _

    lax.fori_loop(0, ACC_R // 16, zero_body, None)

    for k in range(2):
        pltpu.sync_copy(edge.at[pl.ds(k * E + g * ED, ED)], idxs[k])

        def hist_body(j, _, k=k):
            iv = idxs[k][pl.ds(j * 16, 16)]
            plsc.addupdate_scatter(hists[k], [iv], ones16)
            return _

        lax.fori_loop(0, ED // 16, hist_body, None)

    for k in range(2):
        pltpu.sync_copy(hists[k], hist_sh.at[pl.ds((s * 2 + k) * ACC_R, ACC_R)])
    plsc.subcore_barrier()

    for k in range(2):
        for t in range(NS):
            pltpu.sync_copy(
                hist_sh.at[pl.ds((t * 2 + k) * ACC_R + s * SL, SL)],
                rbuf.at[pl.ds(t * SL, SL)],
            )

        def red_body(j, _):
            acc = rbuf[pl.ds(j * 16, 16)]
            for t in range(1, NS):
                acc = acc + rbuf[pl.ds(t * SL + j * 16, 16)]
            obuf[pl.ds(j * 16, 16)] = acc
            return _

        lax.fori_loop(0, SL // 16, red_body, None)
        pltpu.sync_copy(obuf, parts.at[pl.ds((c * 2 + k) * ACC_R + s * SL, SL)])


def _deg_call(edge_flat):
    return pl.kernel(
        _deg_body,
        out_type=jax.ShapeDtypeStruct((NC * 2 * ACC_R,), jnp.float32),
        mesh=_mesh(),
        compiler_params=_sc_params,
        scratch_types=[
            pltpu.VMEM((ACC_R,), jnp.float32),
            pltpu.VMEM((ACC_R,), jnp.float32),
            pltpu.VMEM((ED,), jnp.int32),
            pltpu.VMEM((ED,), jnp.int32),
            pltpu.VMEM_SHARED((NS * 2 * ACC_R,), jnp.float32),
            pltpu.VMEM((NS * SL,), jnp.float32),
            pltpu.VMEM((SL,), jnp.float32),
        ],
    )(edge_flat)


_NB = 3


def _agg_body(table, srcp, dstp, zeros, y, acc, *bufs):
    si = bufs[0:_NB]
    di = bufs[_NB:2 * _NB]
    rows = bufs[2 * _NB:3 * _NB]
    gs = bufs[3 * _NB:4 * _NB]
    ss = bufs[4 * _NB:5 * _NB]
    c = lax.axis_index("c")
    s = lax.axis_index("s")

    zdesc = pltpu.async_copy(
        zeros.at[pl.ds(s * ASL, ASL)], acc.at[pl.ds(s * ASL, ASL)], ss[0]
    )

    base = (c * NS + s) * EPT

    def load_idx(g, j):
        e0 = pl.multiple_of(base + g * C, 8)
        pltpu.sync_copy(srcp.at[pl.ds(e0, C)], si[j])
        pltpu.sync_copy(dstp.at[pl.ds(e0, C)], di[j])

    for j in range(_NB):
        load_idx(j, j)
        pltpu.async_copy(table.at[si[j]], rows[j], gs[j])

    zdesc.wait()
    plsc.subcore_barrier()

    def triad_body(q, _):
        descs = []
        for j in range(_NB):
            pltpu.make_async_copy(table.at[si[j]], rows[j], gs[j]).wait()
            descs.append(pltpu.async_copy(rows[j], acc.at[di[j]], ss[j], add=True))
        g0 = jnp.minimum((q + 1) * _NB, CH - _NB)
        for j in range(_NB):
            descs[j].wait()
            load_idx(g0 + j, j)
            pltpu.async_copy(table.at[si[j]], rows[j], gs[j])
        return _

    lax.fori_loop(0, CH // _NB, triad_body, None)

    for j in range(_NB):
        pltpu.make_async_copy(table.at[si[j]], rows[j], gs[j]).wait()

    plsc.subcore_barrier()
    pltpu.sync_copy(acc.at[pl.ds(s * ASL, ASL)], y.at[c, pl.ds(s * ASL, ASL)])


def _agg_call(table, srcp, dstp, zeros):
    return pl.kernel(
        _agg_body,
        out_type=jax.ShapeDtypeStruct((NC, AGG_R, D), jnp.float32),
        mesh=_mesh(),
        compiler_params=_sc_params,
        scratch_types=[
            pltpu.VMEM_SHARED((AGG_R, D), jnp.float32),
            *[pltpu.VMEM((C,), jnp.int32) for _ in range(_NB)],
            *[pltpu.VMEM((C,), jnp.int32) for _ in range(_NB)],
            *[pltpu.VMEM((C, D), jnp.float32) for _ in range(_NB)],
            *[pltpu.SemaphoreType.DMA for _ in range(_NB)],
            *[pltpu.SemaphoreType.DMA for _ in range(_NB)],
        ],
    )(table, srcp, dstp, zeros)


_ROWS_BLK = 1000


def _tc1_body(x_ref, no_ref, w_ref, o_ref):
    o_ref[...] = (x_ref[...] * no_ref[...]) @ w_ref[...]


def _tc2_body(y_ref, ni_ref, no_ref, b1_ref, w2_ref, o_ref):
    yb = y_ref[0] + y_ref[1]
    h = jnp.maximum(yb * ni_ref[...] + b1_ref[...], 0.0)
    o_ref[...] = (h * no_ref[...]) @ w2_ref[...]


def _tc3_body(y_ref, ni_ref, b2_ref, o_ref):
    yb = y_ref[0] + y_ref[1]
    o_ref[...] = yb * ni_ref[...] + b2_ref[...]


def _tc1(x, no2, W1):
    return pl.pallas_call(
        _tc1_body,
        out_shape=jax.ShapeDtypeStruct((N, D), jnp.float32),
        grid=(N // _ROWS_BLK,),
        in_specs=[
            pl.BlockSpec((_ROWS_BLK, D), lambda i: (i, 0)),
            pl.BlockSpec((_ROWS_BLK, 1), lambda i: (i, 0)),
            pl.BlockSpec((D, D), lambda i: (0, 0)),
        ],
        out_specs=pl.BlockSpec((_ROWS_BLK, D), lambda i: (i, 0)),
    )(x, no2, W1)


def _tc2(y1, ni2, no2, b1, W2):
    return pl.pallas_call(
        _tc2_body,
        out_shape=jax.ShapeDtypeStruct((N, D), jnp.float32),
        grid=(N // _ROWS_BLK,),
        in_specs=[
            pl.BlockSpec((NC, _ROWS_BLK, D), lambda i: (0, i, 0)),
            pl.BlockSpec((_ROWS_BLK, 1), lambda i: (i, 0)),
            pl.BlockSpec((_ROWS_BLK, 1), lambda i: (i, 0)),
            pl.BlockSpec((1, D), lambda i: (0, 0)),
            pl.BlockSpec((D, D), lambda i: (0, 0)),
        ],
        out_specs=pl.BlockSpec((_ROWS_BLK, D), lambda i: (i, 0)),
    )(y1, ni2, no2, b1, W2)


def _tc3(y2, ni2, b2):
    return pl.pallas_call(
        _tc3_body,
        out_shape=jax.ShapeDtypeStruct((N, D), jnp.float32),
        grid=(N // _ROWS_BLK,),
        in_specs=[
            pl.BlockSpec((NC, _ROWS_BLK, D), lambda i: (0, i, 0)),
            pl.BlockSpec((_ROWS_BLK, 1), lambda i: (i, 0)),
            pl.BlockSpec((1, D), lambda i: (0, 0)),
        ],
        out_specs=pl.BlockSpec((_ROWS_BLK, D), lambda i: (i, 0)),
    )(y2, ni2, b2)


@jax.jit
def kernel(x, edge_index, W1, b1, W2, b2):
    src = edge_index[0]
    dst = edge_index[1]

    parts = _deg_call(edge_index.reshape(-1)).reshape(NC, 2, ACC_R)
    deg = parts[0] + parts[1]
    norm_out2 = lax.rsqrt(jnp.clip(deg[0, :N], 1.0, None))[:, None]
    norm_in2 = lax.rsqrt(jnp.clip(deg[1, :N], 1.0, None))[:, None]

    pad = jnp.arange(E_PAD - E, dtype=jnp.int32) % 16
    srcp = jnp.concatenate([src, pad])
    dstp = jnp.concatenate([dst, N + pad])
    zeros = jnp.zeros((AGG_R, D), jnp.float32)

    t1 = _tc1(x, norm_out2, W1)
    y1 = _agg_call(t1, srcp, dstp, zeros)

    t2 = _tc2(y1, norm_in2, norm_out2, b1[None, :], W2)
    y2 = _agg_call(t2, srcp, dstp, zeros)

    return _tc3(y2, norm_in2, b2[None, :])


# speedup vs baseline: 1.0677x; 1.0282x over previous
import functools

import jax
import jax.numpy as jnp
from jax import lax
from jax.experimental import pallas as pl
from jax.experimental.pallas import tpu as pltpu
from jax.experimental.pallas import tpu_sc as plsc

N = 10000
E = 320000
D = 128

NC = 2
NS = 16
NW = NC * NS

ACC_R = 10240
SL = ACC_R // NS
AGG_R = 10112
ASL = AGG_R // NS

C = 128
CH = 81
EPT = C * CH
E_PAD = NW * EPT
ED = E // NW


@functools.cache
def _mesh():
    return plsc.VectorSubcoreMesh(
        core_axis_name="c", subcore_axis_name="s", num_cores=NC, num_subcores=NS
    )


_sc_params = pltpu.CompilerParams(needs_layout_passes=False)


def _deg_body(edge, parts, hist0, hist1, idx0, idx1, hist_sh, rbuf, obuf):
    c = lax.axis_index("c")
    s = lax.axis_index("s")
    g = c * NS + s
    zeros16 = jnp.zeros((16,), jnp.float32)
    ones16 = jnp.ones((16,), jnp.float32)
    hists = (hist0, hist1)
    idxs = (idx0, idx1)

    def zero_body(i, _):
        for u in range(4):
            hist0[pl.ds((i * 4 + u) * 16, 16)] = zeros16
            hist1[pl.ds((i * 4 + u) * 16, 16)] = zeros16
        return _

    lax.fori_loop(0, ACC_R // 64, zero_body, None)

    for k in range(2):
        pltpu.sync_copy(edge.at[pl.ds(k * E + g * ED, ED)], idxs[k])

        def hist_body(j, _, k=k):
            for u in range(5):
                iv = idxs[k][pl.ds((j * 5 + u) * 16, 16)]
                plsc.addupdate_scatter(hists[k], [iv], ones16)
            return _

        lax.fori_loop(0, ED // 80, hist_body, None)

    for k in range(2):
        pltpu.sync_copy(hists[k], hist_sh.at[pl.ds((s * 2 + k) * ACC_R, ACC_R)])
    plsc.subcore_barrier()

    for k in range(2):
        for t in range(NS):
            pltpu.sync_copy(
                hist_sh.at[pl.ds((t * 2 + k) * ACC_R + s * SL, SL)],
                rbuf.at[pl.ds(t * SL, SL)],
            )

        def red_body(j, _):
            acc = rbuf[pl.ds(j * 16, 16)]
            for t in range(1, NS):
                acc = acc + rbuf[pl.ds(t * SL + j * 16, 16)]
            obuf[pl.ds(j * 16, 16)] = acc
            return _

        lax.fori_loop(0, SL // 16, red_body, None)
        pltpu.sync_copy(obuf, parts.at[pl.ds((c * 2 + k) * ACC_R + s * SL, SL)])


def _deg_call(edge_flat):
    return pl.kernel(
        _deg_body,
        out_type=jax.ShapeDtypeStruct((NC * 2 * ACC_R,), jnp.float32),
        mesh=_mesh(),
        compiler_params=_sc_params,
        scratch_types=[
            pltpu.VMEM((ACC_R,), jnp.float32),
            pltpu.VMEM((ACC_R,), jnp.float32),
            pltpu.VMEM((ED,), jnp.int32),
            pltpu.VMEM((ED,), jnp.int32),
            pltpu.VMEM_SHARED((NS * 2 * ACC_R,), jnp.float32),
            pltpu.VMEM((NS * SL,), jnp.float32),
            pltpu.VMEM((SL,), jnp.float32),
        ],
    )(edge_flat)


_NB = 3


def _agg_body(table, srcp, dstp, zeros, y, acc, *bufs):
    si = bufs[0:_NB]
    di = bufs[_NB:2 * _NB]
    rows = bufs[2 * _NB:3 * _NB]
    gs = bufs[3 * _NB:4 * _NB]
    ss = bufs[4 * _NB:5 * _NB]
    c = lax.axis_index("c")
    s = lax.axis_index("s")

    zdesc = pltpu.async_copy(
        zeros.at[pl.ds(s * ASL, ASL)], acc.at[pl.ds(s * ASL, ASL)], ss[0]
    )

    base = (c * NS + s) * EPT

    def load_idx(g, j):
        e0 = pl.multiple_of(base + g * C, 8)
        pltpu.sync_copy(srcp.at[pl.ds(e0, C)], si[j])
        pltpu.sync_copy(dstp.at[pl.ds(e0, C)], di[j])

    for j in range(_NB):
        load_idx(j, j)
        pltpu.async_copy(table.at[si[j]], rows[j], gs[j])

    zdesc.wait()
    plsc.subcore_barrier()

    def triad_body(q, _):
        descs = []
        for j in range(_NB):
            pltpu.make_async_copy(table.at[si[j]], rows[j], gs[j]).wait()
            descs.append(pltpu.async_copy(rows[j], acc.at[di[j]], ss[j], add=True))
        g0 = jnp.minimum((q + 1) * _NB, CH - _NB)
        for j in range(_NB):
            descs[j].wait()
            load_idx(g0 + j, j)
            pltpu.async_copy(table.at[si[j]], rows[j], gs[j])
        return _

    lax.fori_loop(0, CH // _NB, triad_body, None)

    for j in range(_NB):
        pltpu.make_async_copy(table.at[si[j]], rows[j], gs[j]).wait()

    plsc.subcore_barrier()
    pltpu.sync_copy(acc.at[pl.ds(s * ASL, ASL)], y.at[c, pl.ds(s * ASL, ASL)])


def _agg_call(table, srcp, dstp, zeros):
    return pl.kernel(
        _agg_body,
        out_type=jax.ShapeDtypeStruct((NC, AGG_R, D), jnp.float32),
        mesh=_mesh(),
        compiler_params=_sc_params,
        scratch_types=[
            pltpu.VMEM_SHARED((AGG_R, D), jnp.float32),
            *[pltpu.VMEM((C,), jnp.int32) for _ in range(_NB)],
            *[pltpu.VMEM((C,), jnp.int32) for _ in range(_NB)],
            *[pltpu.VMEM((C, D), jnp.float32) for _ in range(_NB)],
            *[pltpu.SemaphoreType.DMA for _ in range(_NB)],
            *[pltpu.SemaphoreType.DMA for _ in range(_NB)],
        ],
    )(table, srcp, dstp, zeros)


_ROWS_BLK = 2000


def _tc1_body(x_ref, no_ref, w_ref, o_ref):
    o_ref[...] = (x_ref[...] * no_ref[...]) @ w_ref[...]


def _tc2_body(y_ref, ni_ref, no_ref, b1_ref, w2_ref, o_ref):
    yb = y_ref[0] + y_ref[1]
    h = jnp.maximum(yb * ni_ref[...] + b1_ref[...], 0.0)
    o_ref[...] = (h * no_ref[...]) @ w2_ref[...]


def _tc3_body(y_ref, ni_ref, b2_ref, o_ref):
    yb = y_ref[0] + y_ref[1]
    o_ref[...] = yb * ni_ref[...] + b2_ref[...]


def _tc1(x, no2, W1):
    return pl.pallas_call(
        _tc1_body,
        out_shape=jax.ShapeDtypeStruct((N, D), jnp.float32),
        grid=(N // _ROWS_BLK,),
        in_specs=[
            pl.BlockSpec((_ROWS_BLK, D), lambda i: (i, 0)),
            pl.BlockSpec((_ROWS_BLK, 1), lambda i: (i, 0)),
            pl.BlockSpec((D, D), lambda i: (0, 0)),
        ],
        out_specs=pl.BlockSpec((_ROWS_BLK, D), lambda i: (i, 0)),
    )(x, no2, W1)


def _tc2(y1, ni2, no2, b1, W2):
    return pl.pallas_call(
        _tc2_body,
        out_shape=jax.ShapeDtypeStruct((N, D), jnp.float32),
        grid=(N // _ROWS_BLK,),
        in_specs=[
            pl.BlockSpec((NC, _ROWS_BLK, D), lambda i: (0, i, 0)),
            pl.BlockSpec((_ROWS_BLK, 1), lambda i: (i, 0)),
            pl.BlockSpec((_ROWS_BLK, 1), lambda i: (i, 0)),
            pl.BlockSpec((1, D), lambda i: (0, 0)),
            pl.BlockSpec((D, D), lambda i: (0, 0)),
        ],
        out_specs=pl.BlockSpec((_ROWS_BLK, D), lambda i: (i, 0)),
    )(y1, ni2, no2, b1, W2)


def _tc3(y2, ni2, b2):
    return pl.pallas_call(
        _tc3_body,
        out_shape=jax.ShapeDtypeStruct((N, D), jnp.float32),
        grid=(N // _ROWS_BLK,),
        in_specs=[
            pl.BlockSpec((NC, _ROWS_BLK, D), lambda i: (0, i, 0)),
            pl.BlockSpec((_ROWS_BLK, 1), lambda i: (i, 0)),
            pl.BlockSpec((1, D), lambda i: (0, 0)),
        ],
        out_specs=pl.BlockSpec((_ROWS_BLK, D), lambda i: (i, 0)),
    )(y2, ni2, b2)


@jax.jit
def kernel(x, edge_index, W1, b1, W2, b2):
    src = edge_index[0]
    dst = edge_index[1]

    parts = _deg_call(edge_index.reshape(-1)).reshape(NC, 2, ACC_R)
    deg = parts[0] + parts[1]
    norm_out2 = lax.rsqrt(jnp.clip(deg[0, :N], 1.0, None))[:, None]
    norm_in2 = lax.rsqrt(jnp.clip(deg[1, :N], 1.0, None))[:, None]

    pad = jnp.arange(E_PAD - E, dtype=jnp.int32) % 16
    srcp = jnp.concatenate([src, pad])
    dstp = jnp.concatenate([dst, N + pad])
    zeros = jnp.zeros((AGG_R, D), jnp.float32)

    t1 = _tc1(x, norm_out2, W1)
    y1 = _agg_call(t1, srcp, dstp, zeros)

    t2 = _tc2(y1, norm_in2, norm_out2, b1[None, :], W2)
    y2 = _agg_call(t2, srcp, dstp, zeros)

    return _tc3(y2, norm_in2, b2[None, :])
